# baseline (speedup 1.0000x reference)
import functools

import jax
import jax.numpy as jnp
from jax import lax
from jax.experimental import pallas as pl
from jax.experimental.pallas import tpu as pltpu
from jax.experimental.pallas import tpu_sc as plsc

_N_NODES = 10000
_N_EDGES = 320000
_D_FEAT = 128
_D_EDGE = 16
_HID = 128

_NC, _NS = 2, 16
_NW = _NC * _NS
_CH = 128
_N_PAD = 10240
_RPT = _N_PAD // _NS

_SLICES = (
    (3600, 28, 16),
    (5120, 40, 0),
    (1280, 10, 0),
)
_EPW_S = 10000
_NFULL_S, _TAIL_S = 78, 16

_EB = 2560
_NB = 1000

_f32 = jnp.float32
_bf16 = jnp.bfloat16

_MESH = dict(core_axis_name="c", subcore_axis_name="s",
             num_cores=_NC, num_subcores=_NS)


def _softplus(v):
    return jnp.maximum(v, 0.0) + jnp.log1p(jnp.exp(-jnp.abs(v)))


def _proj_body(x_ref, ws_ref, wd_ref, xs_ref, xd_ref):
    xv = x_ref[...]
    xs_ref[...] = jnp.dot(xv, ws_ref[...], preferred_element_type=_f32)
    xd_ref[...] = jnp.dot(xv, wd_ref[...], preferred_element_type=_f32)


_proj = pl.pallas_call(
    _proj_body,
    grid=(_N_NODES // _NB,),
    in_specs=[
        pl.BlockSpec((_NB, _D_FEAT), lambda i: (i, 0)),
        pl.BlockSpec((_D_FEAT, _HID), lambda i: (0, 0)),
        pl.BlockSpec((_D_FEAT, _HID), lambda i: (0, 0)),
    ],
    out_specs=[
        pl.BlockSpec((_NB, _HID), lambda i: (i, 0)),
        pl.BlockSpec((_NB, _HID), lambda i: (i, 0)),
    ],
    out_shape=[
        jax.ShapeDtypeStruct((_N_NODES, _HID), _f32),
        jax.ShapeDtypeStruct((_N_NODES, _HID), _f32),
    ],
)


@functools.cache
def _make_gather(epw, nfull, tail):
  def _body(rm, cm, rt, ct, xs, xd, g_out,
            ir_v, ic_v, irt_v, ict_v,
            ba0, bb0, ba1, bb1,
            sa0, sb0, sa1, sb1, w0, w1):
    c = lax.axis_index("c")
    s = lax.axis_index("s")
    wid = s * _NC + c
    base = wid * epw
    pltpu.sync_copy(rm.at[wid], ir_v)
    pltpu.sync_copy(cm.at[wid], ic_v)
    if tail:
        pltpu.sync_copy(rt.at[wid], irt_v)
        pltpu.sync_copy(ct.at[wid], ict_v)

    def _add(dst, src, rows):
        def add_row(r, carry):
            for k in range(_HID // 16):
                sl = pl.ds(k * 16, 16)
                plsc.addupdate(dst.at[r, sl], src[r, sl])
            return carry
        lax.fori_loop(0, rows, add_row, 0)

    def _issue(j, ba, bb, sa, sb):
        pltpu.async_copy(xs.at[ir_v.at[j]], ba, sa)
        pltpu.async_copy(xd.at[ic_v.at[j]], bb, sb)

    def _wait_gathers(j, ba, bb, sa, sb):
        pltpu.make_async_copy(xs.at[ir_v.at[j]], ba, sa).wait()
        pltpu.make_async_copy(xd.at[ic_v.at[j]], bb, sb).wait()

    _issue(0, ba0, bb0, sa0, sb0)
    _issue(1, ba1, bb1, sa1, sb1)

    def body(k, carry):
        j0 = 2 * k
        j1 = 2 * k + 1
        _wait_gathers(j0, ba0, bb0, sa0, sb0)
        _add(ba0, bb0, _CH)
        pltpu.async_copy(ba0, g_out.at[pl.ds(base + j0 * _CH, _CH)], w0)
        _wait_gathers(j1, ba1, bb1, sa1, sb1)
        _add(ba1, bb1, _CH)
        pltpu.async_copy(ba1, g_out.at[pl.ds(base + j1 * _CH, _CH)], w1)
        pltpu.make_async_copy(ba0, g_out.at[pl.ds(base + j0 * _CH, _CH)], w0).wait()
        _issue(j0 + 2, ba0, bb0, sa0, sb0)
        pltpu.make_async_copy(ba1, g_out.at[pl.ds(base + j1 * _CH, _CH)], w1).wait()
        _issue(j1 + 2, ba1, bb1, sa1, sb1)
        return carry

    lax.fori_loop(0, nfull // 2 - 1, body, 0)

    j0 = nfull - 2
    j1 = nfull - 1
    _wait_gathers(j0, ba0, bb0, sa0, sb0)
    _add(ba0, bb0, _CH)
    pltpu.sync_copy(ba0, g_out.at[pl.ds(base + j0 * _CH, _CH)])
    _wait_gathers(j1, ba1, bb1, sa1, sb1)
    _add(ba1, bb1, _CH)
    pltpu.sync_copy(ba1, g_out.at[pl.ds(base + j1 * _CH, _CH)])

    if tail:
        tbase = base + nfull * _CH
        cpa = pltpu.async_copy(xs.at[irt_v], ba0.at[pl.ds(0, tail)], sa0)
        cpb = pltpu.async_copy(xd.at[ict_v], bb0.at[pl.ds(0, tail)], sb0)
        cpa.wait()
        cpb.wait()
        _add(ba0, bb0, tail)
        pltpu.sync_copy(ba0.at[pl.ds(0, tail)], g_out.at[pl.ds(tbase, tail)])

  return pl.kernel(
      _body,
      out_type=jax.ShapeDtypeStruct((_NW * epw, _HID), _f32),
      mesh=plsc.VectorSubcoreMesh(**_MESH),
      scratch_types=[
          pltpu.VMEM((nfull, _CH), jnp.int32),
          pltpu.VMEM((nfull, _CH), jnp.int32),
          pltpu.VMEM((max(tail, 8),), jnp.int32),
          pltpu.VMEM((max(tail, 8),), jnp.int32),
          pltpu.VMEM((_CH, _HID), _f32),
          pltpu.VMEM((_CH, _HID), _f32),
          pltpu.VMEM((_CH, _HID), _f32),
          pltpu.VMEM((_CH, _HID), _f32),
          pltpu.SemaphoreType.DMA,
          pltpu.SemaphoreType.DMA,
          pltpu.SemaphoreType.DMA,
          pltpu.SemaphoreType.DMA,
          pltpu.SemaphoreType.DMA,
          pltpu.SemaphoreType.DMA,
      ],
  )


def _edge_math(g_ref, eat_ref, w1e_ref, b1_ref, w2_ref, b2_ref, out_ref):
    ea_term = lax.dot_general(eat_ref[...].astype(_bf16), w1e_ref[...],
                              (((0,), (0,)), ((), ())),
                              preferred_element_type=_f32)
    pre = g_ref[...] + ea_term + b1_ref[...]
    h = _softplus(pre)
    out_ref[...] = jnp.dot(h.astype(_bf16), w2_ref[...],
                           preferred_element_type=_f32) + b2_ref[...]


def _edge_body2(e_ref, g_ref, eat_ref, w1e_ref, b1_ref, w2_ref, b2_ref, out_ref):
    del e_ref
    _edge_math(g_ref, eat_ref, w1e_ref, b1_ref, w2_ref, b2_ref, out_ref)


_W_SPECS = [
    pl.BlockSpec((_D_EDGE, _HID), lambda i: (0, 0)),
    pl.BlockSpec((1, _HID), lambda i: (0, 0)),
    pl.BlockSpec((_HID, _HID), lambda i: (0, 0)),
    pl.BlockSpec((1, _HID), lambda i: (0, 0)),
]


@functools.cache
def _make_edge(nblocks, boff, aliased):
    gspec = pl.BlockSpec((_EB, _HID), lambda i: (i, 0))
    easpec = pl.BlockSpec((_D_EDGE, _EB), lambda i, o=boff: (0, i + o))
    ospec = pl.BlockSpec((_EB, _HID), lambda i, o=boff: (i + o, 0))
    oshape = jax.ShapeDtypeStruct((_N_EDGES, _HID), _f32)
    if not aliased:
        return pl.pallas_call(
            _edge_math, grid=(nblocks,),
            in_specs=[gspec, easpec, *_W_SPECS],
            out_specs=ospec, out_shape=oshape)
    return pl.pallas_call(
        _edge_body2, grid=(nblocks,),
        in_specs=[pl.BlockSpec(memory_space=pltpu.MemorySpace.HBM),
                  gspec, easpec, *_W_SPECS],
        out_specs=ospec, out_shape=oshape,
        input_output_aliases={0: 0})


@functools.cache
def _make_scatter(epw, nfull, tail, base_off):
  def _body(rm, rt, e_in, out, ir_v, irt_v, eb0, eb1, tbuf, acc, s0, s1):
    c = lax.axis_index("c")
    s = lax.axis_index("s")
    wid = s * _NC + c
    base = base_off + wid * epw
    pltpu.sync_copy(rm.at[wid], ir_v)
    if tail:
        pltpu.sync_copy(rt.at[wid], irt_v)

    def zrow(r, carry):
        for k in range(_HID // 16):
            eb0[r, pl.ds(k * 16, 16)] = jnp.zeros((16,), _f32)
        return carry

    lax.fori_loop(0, _CH, zrow, 0)
    row0 = s * _RPT
    for t in range(_RPT // _CH):
        pltpu.sync_copy(eb0, acc.at[pl.ds(row0 + t * _CH, _CH)])
    plsc.subcore_barrier()

    pltpu.async_copy(e_in.at[pl.ds(base, _CH)], eb0, s0)
    pltpu.async_copy(e_in.at[pl.ds(base + _CH, _CH)], eb1, s1)

    def step(k, carry):
        j0 = 2 * k
        j1 = 2 * k + 1
        pltpu.make_async_copy(e_in.at[pl.ds(base + j0 * _CH, _CH)], eb0, s0).wait()
        pltpu.sync_copy(eb0, acc.at[ir_v.at[j0]], add=True)
        pltpu.async_copy(e_in.at[pl.ds(base + (j0 + 2) * _CH, _CH)], eb0, s0)
        pltpu.make_async_copy(e_in.at[pl.ds(base + j1 * _CH, _CH)], eb1, s1).wait()
        pltpu.sync_copy(eb1, acc.at[ir_v.at[j1]], add=True)
        pltpu.async_copy(e_in.at[pl.ds(base + (j1 + 2) * _CH, _CH)], eb1, s1)
        return carry

    lax.fori_loop(0, nfull // 2 - 1, step, 0)
    j0 = nfull - 2
    j1 = nfull - 1
    pltpu.make_async_copy(e_in.at[pl.ds(base + j0 * _CH, _CH)], eb0, s0).wait()
    pltpu.sync_copy(eb0, acc.at[ir_v.at[j0]], add=True)
    pltpu.make_async_copy(e_in.at[pl.ds(base + j1 * _CH, _CH)], eb1, s1).wait()
    pltpu.sync_copy(eb1, acc.at[ir_v.at[j1]], add=True)
    if tail:
        pltpu.sync_copy(e_in.at[pl.ds(base + nfull * _CH, tail)], tbuf)
        pltpu.sync_copy(tbuf, acc.at[irt_v], add=True)
    plsc.subcore_barrier()
    pltpu.sync_copy(acc.at[pl.ds(row0, _RPT)], out.at[c, pl.ds(row0, _RPT)])

  return pl.kernel(
      _body,
      out_type=jax.ShapeDtypeStruct((_NC, _N_PAD, _HID), _f32),
      mesh=plsc.VectorSubcoreMesh(**_MESH),
      scratch_types=[
          pltpu.VMEM((nfull, _CH), jnp.int32),
          pltpu.VMEM((max(tail, 8),), jnp.int32),
          pltpu.VMEM((_CH, _HID), _f32),
          pltpu.VMEM((_CH, _HID), _f32),
          pltpu.VMEM((max(tail, 8), _HID), _f32),
          pltpu.VMEM_SHARED((_N_PAD, _HID), _f32),
          pltpu.SemaphoreType.DMA,
          pltpu.SemaphoreType.DMA,
      ],
  )


_NGRID = _N_NODES // _NB


def _node_body(x_ref, m1_ref, gs_ref, w1x_ref, w1m_ref, b1_ref, w2_ref,
               b2_ref, g1x_ref, g1e_ref, g1g_ref, gb1_ref, g2_ref, gb2_ref,
               xu_ref, gu_ref, accx, accm):
    i = pl.program_id(0)
    msg = m1_ref[0] + m1_ref[1]
    pre = (jnp.dot(x_ref[...], w1x_ref[...], preferred_element_type=_f32)
           + jnp.dot(msg, w1m_ref[...], preferred_element_type=_f32)
           + b1_ref[...])
    h = _softplus(pre)
    xu = jnp.dot(h, w2_ref[...], preferred_element_type=_f32) + b2_ref[...]
    xu_ref[...] = xu
    bx = jnp.sum(xu, axis=0, keepdims=True)
    bm = jnp.sum(msg, axis=0, keepdims=True)

    @pl.when(i == 0)
    def _():
        accx[...] = bx
        accm[...] = bm

    @pl.when(i > 0)
    def _():
        accx[...] = accx[...] + bx
        accm[...] = accm[...] + bm

    @pl.when(i == _NGRID - 1)
    def _():
        mx = accx[...] * (1.0 / _N_NODES)
        me = accm[...] * (1.0 / _N_EDGES)
        gpre = (jnp.dot(mx, g1x_ref[...], preferred_element_type=_f32)
                + jnp.dot(me, g1e_ref[...], preferred_element_type=_f32)
                + jnp.dot(gs_ref[...], g1g_ref[...], preferred_element_type=_f32)
                + gb1_ref[...])
        gh = _softplus(gpre)
        gu_ref[...] = jnp.dot(gh, g2_ref[...], preferred_element_type=_f32) + gb2_ref[...]


_node = pl.pallas_call(
    _node_body,
    grid=(_NGRID,),
    in_specs=[
        pl.BlockSpec((_NB, _D_FEAT), lambda i: (i, 0)),
        pl.BlockSpec((_NC, _NB, _HID), lambda i: (0, i, 0)),
        pl.BlockSpec((1, _HID), lambda i: (0, 0)),
        pl.BlockSpec((_D_FEAT, _HID), lambda i: (0, 0)),
        pl.BlockSpec((_HID, _HID), lambda i: (0, 0)),
        pl.BlockSpec((1, _HID), lambda i: (0, 0)),
        pl.BlockSpec((_HID, _HID), lambda i: (0, 0)),
        pl.BlockSpec((1, _HID), lambda i: (0, 0)),
        pl.BlockSpec((_HID, _HID), lambda i: (0, 0)),
        pl.BlockSpec((_HID, _HID), lambda i: (0, 0)),
        pl.BlockSpec((_HID, _HID), lambda i: (0, 0)),
        pl.BlockSpec((1, _HID), lambda i: (0, 0)),
        pl.BlockSpec((_HID, _HID), lambda i: (0, 0)),
        pl.BlockSpec((1, _HID), lambda i: (0, 0)),
    ],
    out_specs=[
        pl.BlockSpec((_NB, _HID), lambda i: (i, 0)),
        pl.BlockSpec((1, _HID), lambda i: (0, 0)),
    ],
    out_shape=[
        jax.ShapeDtypeStruct((_N_NODES, _HID), _f32),
        jax.ShapeDtypeStruct((1, _HID), _f32),
    ],
    scratch_shapes=[
        pltpu.VMEM((1, _HID), _f32),
        pltpu.VMEM((1, _HID), _f32),
    ],
)


def kernel(x, edge_index, edge_attr, global_state, eW1, eb1, eW2, eb2,
           nW1, nb1, nW2, nb2, gW1, gb1, gW2, gb2):
    row = edge_index[0].astype(jnp.int32)
    col = edge_index[1].astype(jnp.int32)
    dummy = jnp.zeros((_NW, 8), jnp.int32)

    eW1e = eW1[:_D_EDGE]
    eW1s = eW1[_D_EDGE:_D_EDGE + _D_FEAT]
    eW1d = eW1[_D_EDGE + _D_FEAT:]

    xs, xd = _proj(x, eW1s, eW1d)

    ea_t = edge_attr.T
    ew = (eW1e.astype(_bf16), eb1.reshape(1, _HID), eW2.astype(_bf16),
          eb2.reshape(1, _HID))

    gs_list = []
    off = 0
    for epw, nfull, tail in _SLICES:
        h = _NW * epw
        r = row[off:off + h].reshape(_NW, epw)
        c = col[off:off + h].reshape(_NW, epw)
        rm = r[:, : nfull * _CH].reshape(_NW, nfull, _CH)
        cm = c[:, : nfull * _CH].reshape(_NW, nfull, _CH)
        rt = r[:, nfull * _CH:] if tail else dummy
        ct = c[:, nfull * _CH:] if tail else dummy
        gs_list.append(_make_gather(epw, nfull, tail)(rm, cm, rt, ct, xs, xd))
        off += h

    e_out = None
    boff = 0
    for (epw, nfull, tail), g in zip(_SLICES, gs_list):
        nblocks = _NW * epw // _EB
        if e_out is None:
            e_out = _make_edge(nblocks, boff, False)(g, ea_t, *ew)
        else:
            e_out = _make_edge(nblocks, boff, True)(e_out, g, ea_t, *ew)
        boff += nblocks

    rs = row.reshape(_NW, _EPW_S)
    rms = rs[:, : _NFULL_S * _CH].reshape(_NW, _NFULL_S, _CH)
    rts = rs[:, _NFULL_S * _CH:]
    msgp = _make_scatter(_EPW_S, _NFULL_S, _TAIL_S, 0)(rms, rts, e_out)

    xu, gu = _node(x, msgp, global_state,
                   nW1[:_D_FEAT], nW1[_D_FEAT:], nb1.reshape(1, _HID),
                   nW2, nb2.reshape(1, _HID),
                   gW1[:_HID], gW1[_HID:2 * _HID], gW1[2 * _HID:],
                   gb1.reshape(1, _HID), gW2, gb2.reshape(1, _HID))
    return (xu, e_out, gu)


# speedup vs baseline: 1.0187x; 1.0187x over previous
import functools

import jax
import jax.numpy as jnp
from jax import lax
from jax.experimental import pallas as pl
from jax.experimental.pallas import tpu as pltpu
from jax.experimental.pallas import tpu_sc as plsc

_N_NODES = 10000
_N_EDGES = 320000
_D_FEAT = 128
_D_EDGE = 16
_HID = 128

_NC, _NS = 2, 16
_NW = _NC * _NS
_CH = 128
_N_PAD = 10240
_RPT = _N_PAD // _NS

_SLICES = (
    (2560, 20, 0),
    (3600, 28, 16),
    (3840, 30, 0),
)
_EPW_S = 10000
_NFULL_S, _TAIL_S = 78, 16

_EB = 2560
_NB = 1000

_f32 = jnp.float32
_bf16 = jnp.bfloat16

_MESH = dict(core_axis_name="c", subcore_axis_name="s",
             num_cores=_NC, num_subcores=_NS)


def _softplus(v):
    return jnp.maximum(v, 0.0) + jnp.log1p(jnp.exp(-jnp.abs(v)))


def _proj_body(x_ref, ws_ref, wd_ref, xs_ref, xd_ref):
    xv = x_ref[...]
    xs_ref[...] = jnp.dot(xv, ws_ref[...], preferred_element_type=_f32)
    xd_ref[...] = jnp.dot(xv, wd_ref[...], preferred_element_type=_f32)


_proj = pl.pallas_call(
    _proj_body,
    grid=(_N_NODES // _NB,),
    in_specs=[
        pl.BlockSpec((_NB, _D_FEAT), lambda i: (i, 0)),
        pl.BlockSpec((_D_FEAT, _HID), lambda i: (0, 0)),
        pl.BlockSpec((_D_FEAT, _HID), lambda i: (0, 0)),
    ],
    out_specs=[
        pl.BlockSpec((_NB, _HID), lambda i: (i, 0)),
        pl.BlockSpec((_NB, _HID), lambda i: (i, 0)),
    ],
    out_shape=[
        jax.ShapeDtypeStruct((_N_NODES, _HID), _f32),
        jax.ShapeDtypeStruct((_N_NODES, _HID), _f32),
    ],
)


@functools.cache
def _make_gather(epw, nfull, tail):
  def _body(rm, cm, rt, ct, xs, xd, g_out,
            ir_v, ic_v, irt_v, ict_v,
            ba0, bb0, ba1, bb1,
            sa0, sb0, sa1, sb1, w0, w1):
    c = lax.axis_index("c")
    s = lax.axis_index("s")
    wid = s * _NC + c
    base = wid * epw
    pltpu.sync_copy(rm.at[wid], ir_v)
    pltpu.sync_copy(cm.at[wid], ic_v)
    if tail:
        pltpu.sync_copy(rt.at[wid], irt_v)
        pltpu.sync_copy(ct.at[wid], ict_v)

    def _add(dst, src, rows):
        def add_row(r, carry):
            for k in range(_HID // 16):
                sl = pl.ds(k * 16, 16)
                plsc.addupdate(dst.at[r, sl], src[r, sl])
            return carry
        lax.fori_loop(0, rows, add_row, 0)

    def _issue(j, ba, bb, sa, sb):
        pltpu.async_copy(xs.at[ir_v.at[j]], ba, sa)
        pltpu.async_copy(xd.at[ic_v.at[j]], bb, sb)

    def _wait_gathers(j, ba, bb, sa, sb):
        pltpu.make_async_copy(xs.at[ir_v.at[j]], ba, sa).wait()
        pltpu.make_async_copy(xd.at[ic_v.at[j]], bb, sb).wait()

    _issue(0, ba0, bb0, sa0, sb0)
    _issue(1, ba1, bb1, sa1, sb1)

    def body(k, carry):
        j0 = 2 * k
        j1 = 2 * k + 1
        _wait_gathers(j0, ba0, bb0, sa0, sb0)
        _add(ba0, bb0, _CH)
        pltpu.async_copy(ba0, g_out.at[pl.ds(base + j0 * _CH, _CH)], w0)
        _wait_gathers(j1, ba1, bb1, sa1, sb1)
        _add(ba1, bb1, _CH)
        pltpu.async_copy(ba1, g_out.at[pl.ds(base + j1 * _CH, _CH)], w1)
        pltpu.make_async_copy(ba0, g_out.at[pl.ds(base + j0 * _CH, _CH)], w0).wait()
        _issue(j0 + 2, ba0, bb0, sa0, sb0)
        pltpu.make_async_copy(ba1, g_out.at[pl.ds(base + j1 * _CH, _CH)], w1).wait()
        _issue(j1 + 2, ba1, bb1, sa1, sb1)
        return carry

    lax.fori_loop(0, nfull // 2 - 1, body, 0)

    j0 = nfull - 2
    j1 = nfull - 1
    _wait_gathers(j0, ba0, bb0, sa0, sb0)
    _add(ba0, bb0, _CH)
    pltpu.sync_copy(ba0, g_out.at[pl.ds(base + j0 * _CH, _CH)])
    _wait_gathers(j1, ba1, bb1, sa1, sb1)
    _add(ba1, bb1, _CH)
    pltpu.sync_copy(ba1, g_out.at[pl.ds(base + j1 * _CH, _CH)])

    if tail:
        tbase = base + nfull * _CH
        cpa = pltpu.async_copy(xs.at[irt_v], ba0.at[pl.ds(0, tail)], sa0)
        cpb = pltpu.async_copy(xd.at[ict_v], bb0.at[pl.ds(0, tail)], sb0)
        cpa.wait()
        cpb.wait()
        _add(ba0, bb0, tail)
        pltpu.sync_copy(ba0.at[pl.ds(0, tail)], g_out.at[pl.ds(tbase, tail)])

  return pl.kernel(
      _body,
      out_type=jax.ShapeDtypeStruct((_NW * epw, _HID), _f32),
      mesh=plsc.VectorSubcoreMesh(**_MESH),
      scratch_types=[
          pltpu.VMEM((nfull, _CH), jnp.int32),
          pltpu.VMEM((nfull, _CH), jnp.int32),
          pltpu.VMEM((max(tail, 8),), jnp.int32),
          pltpu.VMEM((max(tail, 8),), jnp.int32),
          pltpu.VMEM((_CH, _HID), _f32),
          pltpu.VMEM((_CH, _HID), _f32),
          pltpu.VMEM((_CH, _HID), _f32),
          pltpu.VMEM((_CH, _HID), _f32),
          pltpu.SemaphoreType.DMA,
          pltpu.SemaphoreType.DMA,
          pltpu.SemaphoreType.DMA,
          pltpu.SemaphoreType.DMA,
          pltpu.SemaphoreType.DMA,
          pltpu.SemaphoreType.DMA,
      ],
  )


def _edge_math(g_ref, eat_ref, w1e_ref, b1_ref, w2_ref, b2_ref, out_ref):
    ea_term = lax.dot_general(eat_ref[...].astype(_bf16), w1e_ref[...],
                              (((0,), (0,)), ((), ())),
                              preferred_element_type=_f32)
    pre = g_ref[...] + ea_term + b1_ref[...]
    h = _softplus(pre)
    out_ref[...] = jnp.dot(h.astype(_bf16), w2_ref[...],
                           preferred_element_type=_f32) + b2_ref[...]


def _edge_body2(e_ref, g_ref, eat_ref, w1e_ref, b1_ref, w2_ref, b2_ref, out_ref):
    del e_ref
    _edge_math(g_ref, eat_ref, w1e_ref, b1_ref, w2_ref, b2_ref, out_ref)


_W_SPECS = [
    pl.BlockSpec((_D_EDGE, _HID), lambda i: (0, 0)),
    pl.BlockSpec((1, _HID), lambda i: (0, 0)),
    pl.BlockSpec((_HID, _HID), lambda i: (0, 0)),
    pl.BlockSpec((1, _HID), lambda i: (0, 0)),
]


@functools.cache
def _make_edge(nblocks, boff, aliased):
    gspec = pl.BlockSpec((_EB, _HID), lambda i: (i, 0))
    easpec = pl.BlockSpec((_D_EDGE, _EB), lambda i, o=boff: (0, i + o))
    ospec = pl.BlockSpec((_EB, _HID), lambda i, o=boff: (i + o, 0))
    oshape = jax.ShapeDtypeStruct((_N_EDGES, _HID), _f32)
    if not aliased:
        return pl.pallas_call(
            _edge_math, grid=(nblocks,),
            in_specs=[gspec, easpec, *_W_SPECS],
            out_specs=ospec, out_shape=oshape)
    return pl.pallas_call(
        _edge_body2, grid=(nblocks,),
        in_specs=[pl.BlockSpec(memory_space=pltpu.MemorySpace.HBM),
                  gspec, easpec, *_W_SPECS],
        out_specs=ospec, out_shape=oshape,
        input_output_aliases={0: 0})


@functools.cache
def _make_scatter(epw, nfull, tail, base_off):
  def _body(rm, rt, e_in, out, ir_v, irt_v, eb0, eb1, tbuf, acc, s0, s1):
    c = lax.axis_index("c")
    s = lax.axis_index("s")
    wid = s * _NC + c
    base = base_off + wid * epw
    pltpu.sync_copy(rm.at[wid], ir_v)
    if tail:
        pltpu.sync_copy(rt.at[wid], irt_v)

    def zrow(r, carry):
        for k in range(_HID // 16):
            eb0[r, pl.ds(k * 16, 16)] = jnp.zeros((16,), _f32)
        return carry

    lax.fori_loop(0, _CH, zrow, 0)
    row0 = s * _RPT
    for t in range(_RPT // _CH):
        pltpu.sync_copy(eb0, acc.at[pl.ds(row0 + t * _CH, _CH)])
    plsc.subcore_barrier()

    pltpu.async_copy(e_in.at[pl.ds(base, _CH)], eb0, s0)
    pltpu.async_copy(e_in.at[pl.ds(base + _CH, _CH)], eb1, s1)

    def step(k, carry):
        j0 = 2 * k
        j1 = 2 * k + 1
        pltpu.make_async_copy(e_in.at[pl.ds(base + j0 * _CH, _CH)], eb0, s0).wait()
        pltpu.sync_copy(eb0, acc.at[ir_v.at[j0]], add=True)
        pltpu.async_copy(e_in.at[pl.ds(base + (j0 + 2) * _CH, _CH)], eb0, s0)
        pltpu.make_async_copy(e_in.at[pl.ds(base + j1 * _CH, _CH)], eb1, s1).wait()
        pltpu.sync_copy(eb1, acc.at[ir_v.at[j1]], add=True)
        pltpu.async_copy(e_in.at[pl.ds(base + (j1 + 2) * _CH, _CH)], eb1, s1)
        return carry

    lax.fori_loop(0, nfull // 2 - 1, step, 0)
    j0 = nfull - 2
    j1 = nfull - 1
    pltpu.make_async_copy(e_in.at[pl.ds(base + j0 * _CH, _CH)], eb0, s0).wait()
    pltpu.sync_copy(eb0, acc.at[ir_v.at[j0]], add=True)
    pltpu.make_async_copy(e_in.at[pl.ds(base + j1 * _CH, _CH)], eb1, s1).wait()
    pltpu.sync_copy(eb1, acc.at[ir_v.at[j1]], add=True)
    if tail:
        pltpu.sync_copy(e_in.at[pl.ds(base + nfull * _CH, tail)], tbuf)
        pltpu.sync_copy(tbuf, acc.at[irt_v], add=True)
    plsc.subcore_barrier()
    pltpu.sync_copy(acc.at[pl.ds(row0, _RPT)], out.at[c, pl.ds(row0, _RPT)])

  return pl.kernel(
      _body,
      out_type=jax.ShapeDtypeStruct((_NC, _N_PAD, _HID), _f32),
      mesh=plsc.VectorSubcoreMesh(**_MESH),
      scratch_types=[
          pltpu.VMEM((nfull, _CH), jnp.int32),
          pltpu.VMEM((max(tail, 8),), jnp.int32),
          pltpu.VMEM((_CH, _HID), _f32),
          pltpu.VMEM((_CH, _HID), _f32),
          pltpu.VMEM((max(tail, 8), _HID), _f32),
          pltpu.VMEM_SHARED((_N_PAD, _HID), _f32),
          pltpu.SemaphoreType.DMA,
          pltpu.SemaphoreType.DMA,
      ],
  )


_NGRID = _N_NODES // _NB


def _node_body(x_ref, m1_ref, gs_ref, w1x_ref, w1m_ref, b1_ref, w2_ref,
               b2_ref, g1x_ref, g1e_ref, g1g_ref, gb1_ref, g2_ref, gb2_ref,
               xu_ref, gu_ref, accx, accm):
    i = pl.program_id(0)
    msg = m1_ref[0] + m1_ref[1]
    pre = (jnp.dot(x_ref[...], w1x_ref[...], preferred_element_type=_f32)
           + jnp.dot(msg, w1m_ref[...], preferred_element_type=_f32)
           + b1_ref[...])
    h = _softplus(pre)
    xu = jnp.dot(h, w2_ref[...], preferred_element_type=_f32) + b2_ref[...]
    xu_ref[...] = xu
    bx = jnp.sum(xu, axis=0, keepdims=True)
    bm = jnp.sum(msg, axis=0, keepdims=True)

    @pl.when(i == 0)
    def _():
        accx[...] = bx
        accm[...] = bm

    @pl.when(i > 0)
    def _():
        accx[...] = accx[...] + bx
        accm[...] = accm[...] + bm

    @pl.when(i == _NGRID - 1)
    def _():
        mx = accx[...] * (1.0 / _N_NODES)
        me = accm[...] * (1.0 / _N_EDGES)
        gpre = (jnp.dot(mx, g1x_ref[...], preferred_element_type=_f32)
                + jnp.dot(me, g1e_ref[...], preferred_element_type=_f32)
                + jnp.dot(gs_ref[...], g1g_ref[...], preferred_element_type=_f32)
                + gb1_ref[...])
        gh = _softplus(gpre)
        gu_ref[...] = jnp.dot(gh, g2_ref[...], preferred_element_type=_f32) + gb2_ref[...]


_node = pl.pallas_call(
    _node_body,
    grid=(_NGRID,),
    in_specs=[
        pl.BlockSpec((_NB, _D_FEAT), lambda i: (i, 0)),
        pl.BlockSpec((_NC, _NB, _HID), lambda i: (0, i, 0)),
        pl.BlockSpec((1, _HID), lambda i: (0, 0)),
        pl.BlockSpec((_D_FEAT, _HID), lambda i: (0, 0)),
        pl.BlockSpec((_HID, _HID), lambda i: (0, 0)),
        pl.BlockSpec((1, _HID), lambda i: (0, 0)),
        pl.BlockSpec((_HID, _HID), lambda i: (0, 0)),
        pl.BlockSpec((1, _HID), lambda i: (0, 0)),
        pl.BlockSpec((_HID, _HID), lambda i: (0, 0)),
        pl.BlockSpec((_HID, _HID), lambda i: (0, 0)),
        pl.BlockSpec((_HID, _HID), lambda i: (0, 0)),
        pl.BlockSpec((1, _HID), lambda i: (0, 0)),
        pl.BlockSpec((_HID, _HID), lambda i: (0, 0)),
        pl.BlockSpec((1, _HID), lambda i: (0, 0)),
    ],
    out_specs=[
        pl.BlockSpec((_NB, _HID), lambda i: (i, 0)),
        pl.BlockSpec((1, _HID), lambda i: (0, 0)),
    ],
    out_shape=[
        jax.ShapeDtypeStruct((_N_NODES, _HID), _f32),
        jax.ShapeDtypeStruct((1, _HID), _f32),
    ],
    scratch_shapes=[
        pltpu.VMEM((1, _HID), _f32),
        pltpu.VMEM((1, _HID), _f32),
    ],
)


def kernel(x, edge_index, edge_attr, global_state, eW1, eb1, eW2, eb2,
           nW1, nb1, nW2, nb2, gW1, gb1, gW2, gb2):
    row = edge_index[0].astype(jnp.int32)
    col = edge_index[1].astype(jnp.int32)
    dummy = jnp.zeros((_NW, 8), jnp.int32)

    eW1e = eW1[:_D_EDGE]
    eW1s = eW1[_D_EDGE:_D_EDGE + _D_FEAT]
    eW1d = eW1[_D_EDGE + _D_FEAT:]

    xs, xd = _proj(x, eW1s, eW1d)

    ea_t = edge_attr.T
    ew = (eW1e.astype(_bf16), eb1.reshape(1, _HID), eW2.astype(_bf16),
          eb2.reshape(1, _HID))

    gs_list = []
    off = 0
    for epw, nfull, tail in _SLICES:
        h = _NW * epw
        r = row[off:off + h].reshape(_NW, epw)
        c = col[off:off + h].reshape(_NW, epw)
        rm = r[:, : nfull * _CH].reshape(_NW, nfull, _CH)
        cm = c[:, : nfull * _CH].reshape(_NW, nfull, _CH)
        rt = r[:, nfull * _CH:] if tail else dummy
        ct = c[:, nfull * _CH:] if tail else dummy
        gs_list.append(_make_gather(epw, nfull, tail)(rm, cm, rt, ct, xs, xd))
        off += h

    e_out = None
    boff = 0
    for (epw, nfull, tail), g in zip(_SLICES, gs_list):
        nblocks = _NW * epw // _EB
        if e_out is None:
            e_out = _make_edge(nblocks, boff, False)(g, ea_t, *ew)
        else:
            e_out = _make_edge(nblocks, boff, True)(e_out, g, ea_t, *ew)
        boff += nblocks

    rs = row.reshape(_NW, _EPW_S)
    rms = rs[:, : _NFULL_S * _CH].reshape(_NW, _NFULL_S, _CH)
    rts = rs[:, _NFULL_S * _CH:]
    msgp = _make_scatter(_EPW_S, _NFULL_S, _TAIL_S, 0)(rms, rts, e_out)

    xu, gu = _node(x, msgp, global_state,
                   nW1[:_D_FEAT], nW1[_D_FEAT:], nb1.reshape(1, _HID),
                   nW2, nb2.reshape(1, _HID),
                   gW1[:_HID], gW1[_HID:2 * _HID], gW1[2 * _HID:],
                   gb1.reshape(1, _HID), gW2, gb2.reshape(1, _HID))
    return (xu, e_out, gu)
